# Initial kernel scaffold; baseline (speedup 1.0000x reference)
import jax
import jax.numpy as jnp
from jax.experimental import pallas as pl


def kernel(wide_features, user_ids, shop_ids, category_ids, numerical_features, wide_W, wide_b, user_table, shop_table, cat_table, W1, b1, W2, b2, W3, b3, Wf, bf):
    raise NotImplementedError("write your pallas kernel here")


# speedup vs baseline: 2.3665x; 2.3665x over previous
import functools

import jax
import jax.numpy as jnp
from jax import lax
from jax.experimental import pallas as pl
from jax.experimental.pallas import tpu as pltpu
from jax.experimental.pallas import tpu_sc as plsc

B = 4096
ED = 128
NU = 10
NUP = 16
H1, H2, H3 = 1024, 512, 256
WIDE = 256

NC, NS = 2, 16
NW = NC * NS
B_PER_W = B // NW

BB = 512
GRID = B // BB


def _sc_gather_body(ut_hbm, st_hbm, ct_hbm, uid_hbm, sid_hbm, cid_hbm,
                    out_u, out_s, out_c,
                    idx_u, idx_s, idx_c, rows_u, rows_s, rows_c, sem):
    wid = lax.axis_index("s") * NC + lax.axis_index("c")
    base = wid * B_PER_W
    pltpu.sync_copy(uid_hbm.at[pl.ds(base, B_PER_W)], idx_u)
    pltpu.sync_copy(sid_hbm.at[pl.ds(base, B_PER_W)], idx_s)
    pltpu.sync_copy(cid_hbm.at[pl.ds(base, B_PER_W)], idx_c)
    g_u = pltpu.make_async_copy(ut_hbm.at[idx_u], rows_u, sem)
    g_s = pltpu.make_async_copy(st_hbm.at[idx_s], rows_s, sem)
    g_c = pltpu.make_async_copy(ct_hbm.at[idx_c], rows_c, sem)
    g_u.start()
    g_s.start()
    g_c.start()
    g_u.wait()
    g_s.wait()
    g_c.wait()
    pltpu.sync_copy(rows_u, out_u.at[pl.ds(base, B_PER_W)])
    pltpu.sync_copy(rows_s, out_s.at[pl.ds(base, B_PER_W)])
    pltpu.sync_copy(rows_c, out_c.at[pl.ds(base, B_PER_W)])


@functools.lru_cache(maxsize=1)
def _sc_gather_kernel():
    return pl.kernel(
        _sc_gather_body,
        out_type=[
            jax.ShapeDtypeStruct((B, ED), jnp.float32),
            jax.ShapeDtypeStruct((B, ED), jnp.float32),
            jax.ShapeDtypeStruct((B, ED), jnp.float32),
        ],
        mesh=plsc.VectorSubcoreMesh(core_axis_name="c",
                                    subcore_axis_name="s",
                                    num_cores=NC, num_subcores=NS),
        scratch_types=[
            pltpu.VMEM((B_PER_W,), jnp.int32),
            pltpu.VMEM((B_PER_W,), jnp.int32),
            pltpu.VMEM((B_PER_W,), jnp.int32),
            pltpu.VMEM((B_PER_W, ED), jnp.float32),
            pltpu.VMEM((B_PER_W, ED), jnp.float32),
            pltpu.VMEM((B_PER_W, ED), jnp.float32),
            pltpu.SemaphoreType.DMA,
        ],
    )


def _mlp_body(u_ref, s_ref, c_ref, n_ref, wide_ref,
              w1u, w1s, w1c, w1n, b1, w2, b2, w3, b3, wW, wfh, wf0, cb,
              out_ref):
    f32 = jnp.float32
    h = jnp.dot(u_ref[:], w1u[:], preferred_element_type=f32)
    h += jnp.dot(s_ref[:], w1s[:], preferred_element_type=f32)
    h += jnp.dot(c_ref[:], w1c[:], preferred_element_type=f32)
    h += jnp.dot(n_ref[:], w1n[:], preferred_element_type=f32)
    h = jnp.maximum(h + b1[:], 0.0)
    h = jnp.maximum(jnp.dot(h, w2[:], preferred_element_type=f32) + b2[:], 0.0)
    h = jnp.maximum(jnp.dot(h, w3[:], preferred_element_type=f32) + b3[:], 0.0)
    wide_dot = jnp.sum(wide_ref[:] * wW[:], axis=1, keepdims=True)
    logit = (jnp.dot(h, wfh[:], preferred_element_type=f32)
             + wide_dot * wf0[:] + cb[:])
    out_ref[:] = 1.0 / (1.0 + jnp.exp(-logit))


def _mlp_call(u_emb, s_emb, c_emb, num_pad, wide_features,
              w1u, w1s, w1c, w1n, b1, w2, b2, w3, b3, wW, wfh, wf0, cb):
    def bspec(cols):
        return pl.BlockSpec((BB, cols), lambda i: (i, 0))

    def wspec(r, c):
        return pl.BlockSpec((r, c), lambda i: (0, 0))

    return pl.pallas_call(
        _mlp_body,
        grid=(GRID,),
        in_specs=[
            bspec(ED), bspec(ED), bspec(ED), bspec(NUP), bspec(WIDE),
            wspec(ED, H1), wspec(ED, H1), wspec(ED, H1), wspec(NUP, H1),
            wspec(1, H1), wspec(H1, H2), wspec(1, H2), wspec(H2, H3),
            wspec(1, H3), wspec(1, WIDE), wspec(H3, 1), wspec(1, 1),
            wspec(1, 1),
        ],
        out_specs=pl.BlockSpec((BB, 1), lambda i: (i, 0)),
        out_shape=jax.ShapeDtypeStruct((B, 1), jnp.float32),
        compiler_params=pltpu.CompilerParams(
            dimension_semantics=("arbitrary",)),
    )(u_emb, s_emb, c_emb, num_pad, wide_features,
      w1u, w1s, w1c, w1n, b1, w2, b2, w3, b3, wW, wfh, wf0, cb)


def kernel(wide_features, user_ids, shop_ids, category_ids,
           numerical_features, wide_W, wide_b, user_table, shop_table,
           cat_table, W1, b1, W2, b2, W3, b3, Wf, bf):
    uid = user_ids.astype(jnp.int32)
    sid = shop_ids.astype(jnp.int32)
    cid = category_ids.astype(jnp.int32)

    u_emb, s_emb, c_emb = _sc_gather_kernel()(
        user_table, shop_table, cat_table, uid, sid, cid)

    num_pad = jnp.pad(numerical_features, ((0, 0), (0, NUP - NU)))
    w1u = W1[:, :ED].T
    w1s = W1[:, ED:2 * ED].T
    w1c = W1[:, 2 * ED:3 * ED].T
    w1n = jnp.pad(W1[:, 3 * ED:], ((0, 0), (0, NUP - NU))).T
    w2 = W2.T
    w3 = W3.T
    wfh = Wf[:, 1:].T
    wf0 = Wf[:, :1]
    cb = (bf + wide_b * Wf[0, 0]).reshape(1, 1)

    return _mlp_call(
        u_emb, s_emb, c_emb, num_pad, wide_features,
        w1u, w1s, w1c, w1n, b1.reshape(1, H1), w2, b2.reshape(1, H2),
        w3, b3.reshape(1, H3), wide_W, wfh, wf0, cb)


# speedup vs baseline: 2.4147x; 1.0203x over previous
import functools

import jax
import jax.numpy as jnp
from jax import lax
from jax.experimental import pallas as pl
from jax.experimental.pallas import tpu as pltpu
from jax.experimental.pallas import tpu_sc as plsc

B = 4096
ED = 128
NU = 10
NUP = 16
H1, H2, H3 = 1024, 512, 256
WIDE = 256

NC, NS = 2, 16
NW = NC * NS
B_PER_W = B // NW

BB = 512
GRID = B // BB


def _sc_gather_body(ut_hbm, st_hbm, ct_hbm, uid_hbm, sid_hbm, cid_hbm,
                    out_u, out_s, out_c,
                    idx_u, idx_s, idx_c, rows_u, rows_s, rows_c, sem):
    wid = lax.axis_index("s") * NC + lax.axis_index("c")
    base = wid * B_PER_W
    pltpu.sync_copy(uid_hbm.at[pl.ds(base, B_PER_W)], idx_u)
    pltpu.sync_copy(sid_hbm.at[pl.ds(base, B_PER_W)], idx_s)
    pltpu.sync_copy(cid_hbm.at[pl.ds(base, B_PER_W)], idx_c)
    g_u = pltpu.make_async_copy(ut_hbm.at[idx_u], rows_u, sem)
    g_s = pltpu.make_async_copy(st_hbm.at[idx_s], rows_s, sem)
    g_c = pltpu.make_async_copy(ct_hbm.at[idx_c], rows_c, sem)
    g_u.start()
    g_s.start()
    g_c.start()
    g_u.wait()
    g_s.wait()
    g_c.wait()
    pltpu.sync_copy(rows_u, out_u.at[pl.ds(base, B_PER_W)])
    pltpu.sync_copy(rows_s, out_s.at[pl.ds(base, B_PER_W)])
    pltpu.sync_copy(rows_c, out_c.at[pl.ds(base, B_PER_W)])


@functools.lru_cache(maxsize=1)
def _sc_gather_kernel():
    return pl.kernel(
        _sc_gather_body,
        out_type=[
            jax.ShapeDtypeStruct((B, ED), jnp.float32),
            jax.ShapeDtypeStruct((B, ED), jnp.float32),
            jax.ShapeDtypeStruct((B, ED), jnp.float32),
        ],
        mesh=plsc.VectorSubcoreMesh(core_axis_name="c",
                                    subcore_axis_name="s",
                                    num_cores=NC, num_subcores=NS),
        scratch_types=[
            pltpu.VMEM((B_PER_W,), jnp.int32),
            pltpu.VMEM((B_PER_W,), jnp.int32),
            pltpu.VMEM((B_PER_W,), jnp.int32),
            pltpu.VMEM((B_PER_W, ED), jnp.float32),
            pltpu.VMEM((B_PER_W, ED), jnp.float32),
            pltpu.VMEM((B_PER_W, ED), jnp.float32),
            pltpu.SemaphoreType.DMA,
        ],
    )


def _mlp_body(u_ref, s_ref, c_ref, n_ref, wide_ref,
              w1u, w1s, w1c, w1n, b1, w2, b2, w3, b3, wW, wfh, wf0, cb,
              out_ref):
    f32 = jnp.float32
    bf16 = jnp.bfloat16
    h = jnp.dot(u_ref[:].astype(bf16), w1u[:], preferred_element_type=f32)
    h += jnp.dot(s_ref[:].astype(bf16), w1s[:], preferred_element_type=f32)
    h += jnp.dot(c_ref[:].astype(bf16), w1c[:], preferred_element_type=f32)
    h += jnp.dot(n_ref[:].astype(bf16), w1n[:], preferred_element_type=f32)
    h = jnp.maximum(h + b1[:], 0.0)
    h = jnp.maximum(
        jnp.dot(h.astype(bf16), w2[:], preferred_element_type=f32) + b2[:],
        0.0)
    h = jnp.maximum(
        jnp.dot(h.astype(bf16), w3[:], preferred_element_type=f32) + b3[:],
        0.0)
    wide_dot = jnp.sum(wide_ref[:] * wW[:], axis=1, keepdims=True)
    logit = (jnp.dot(h.astype(bf16), wfh[:], preferred_element_type=f32)
             + wide_dot * wf0[:] + cb[:])
    out_ref[:] = 1.0 / (1.0 + jnp.exp(-logit))


def _mlp_call(u_emb, s_emb, c_emb, num_pad, wide_features,
              w1u, w1s, w1c, w1n, b1, w2, b2, w3, b3, wW, wfh, wf0, cb):
    def bspec(cols):
        return pl.BlockSpec((BB, cols), lambda i: (i, 0))

    def wspec(r, c):
        return pl.BlockSpec((r, c), lambda i: (0, 0))

    return pl.pallas_call(
        _mlp_body,
        grid=(GRID,),
        in_specs=[
            bspec(ED), bspec(ED), bspec(ED), bspec(NUP), bspec(WIDE),
            wspec(ED, H1), wspec(ED, H1), wspec(ED, H1), wspec(NUP, H1),
            wspec(1, H1), wspec(H1, H2), wspec(1, H2), wspec(H2, H3),
            wspec(1, H3), wspec(1, WIDE), wspec(H3, 1), wspec(1, 1),
            wspec(1, 1),
        ],
        out_specs=pl.BlockSpec((BB, 1), lambda i: (i, 0)),
        out_shape=jax.ShapeDtypeStruct((B, 1), jnp.float32),
        compiler_params=pltpu.CompilerParams(
            dimension_semantics=("arbitrary",)),
    )(u_emb, s_emb, c_emb, num_pad, wide_features,
      w1u, w1s, w1c, w1n, b1, w2, b2, w3, b3, wW, wfh, wf0, cb)


def kernel(wide_features, user_ids, shop_ids, category_ids,
           numerical_features, wide_W, wide_b, user_table, shop_table,
           cat_table, W1, b1, W2, b2, W3, b3, Wf, bf):
    uid = user_ids.astype(jnp.int32)
    sid = shop_ids.astype(jnp.int32)
    cid = category_ids.astype(jnp.int32)

    u_emb, s_emb, c_emb = _sc_gather_kernel()(
        user_table, shop_table, cat_table, uid, sid, cid)

    bf16 = jnp.bfloat16
    num_pad = jnp.pad(numerical_features, ((0, 0), (0, NUP - NU)))
    w1u = W1[:, :ED].T.astype(bf16)
    w1s = W1[:, ED:2 * ED].T.astype(bf16)
    w1c = W1[:, 2 * ED:3 * ED].T.astype(bf16)
    w1n = jnp.pad(W1[:, 3 * ED:], ((0, 0), (0, NUP - NU))).T.astype(bf16)
    w2 = W2.T.astype(bf16)
    w3 = W3.T.astype(bf16)
    wfh = Wf[:, 1:].T.astype(bf16)
    wf0 = Wf[:, :1]
    cb = (bf + wide_b * Wf[0, 0]).reshape(1, 1)

    return _mlp_call(
        u_emb, s_emb, c_emb, num_pad, wide_features,
        w1u, w1s, w1c, w1n, b1.reshape(1, H1), w2, b2.reshape(1, H2),
        w3, b3.reshape(1, H3), wide_W, wfh, wf0, cb)
